# baseline (speedup 1.0000x reference)
import jax
import jax.numpy as jnp
from jax.experimental import pallas as pl
from jax.experimental.pallas import tpu as pltpu

B, S, C = 8, 2048, 64
P, TOPK, DFF = 8, 2, 128


def _moe_body(u_ref, w1_ref, b1_ref, w2t_ref, b2_ref, wr_ref, br_ref, out_ref):
    u = u_ref[0]
    pooled = jnp.mean(u, axis=0, keepdims=True)
    lg = jnp.dot(pooled, wr_ref[...], preferred_element_type=jnp.float32)
    logits = lg[:, :P] + br_ref[0:1, :P]
    iota = jax.lax.broadcasted_iota(jnp.int32, (1, P), 1)
    v1 = jnp.max(logits)
    idx1 = jnp.argmax(logits)
    masked = jnp.where(iota == idx1, -jnp.inf, logits)
    v2 = jnp.max(masked)
    idx2 = jnp.argmax(masked)
    z = jnp.exp(v2 - v1)
    g1 = 1.0 / (1.0 + z)
    g2 = z / (1.0 + z)
    e1 = idx1.astype(jnp.int32)
    e2 = idx2.astype(jnp.int32)
    w1pair = jnp.concatenate([w1_ref[e1], w1_ref[e2]], axis=1)
    b1pair = jnp.concatenate(
        [b1_ref[pl.ds(e1, 1), :], b1_ref[pl.ds(e2, 1), :]], axis=1
    )
    w2pt = jnp.concatenate([g1 * w2t_ref[e1], g2 * w2t_ref[e2]], axis=1)
    b2mix = (
        g1 * b2_ref[pl.ds(e1, 1), :C] + g2 * b2_ref[pl.ds(e2, 1), :C]
    )
    h = jax.nn.gelu(
        jnp.dot(
            u.astype(jnp.bfloat16),
            w1pair.astype(jnp.bfloat16),
            preferred_element_type=jnp.float32,
        )
        + b1pair
    )
    delta = jax.lax.dot_general(
        h.astype(jnp.bfloat16),
        w2pt.astype(jnp.bfloat16),
        (((1,), (1,)), ((), ())),
        preferred_element_type=jnp.float32,
    )
    out_ref[0, :, :C] = u + delta + b2mix
    out_ref[0, :, C:] = jnp.zeros((S, C), jnp.float32)


def kernel(u_t, W1, b1, W2, b2, Wr, br):
    W2T = jnp.swapaxes(W2, 1, 2)
    b2p = jnp.pad(b2, ((0, 0), (0, DFF - C)))
    Wrp = jnp.pad(Wr, ((0, 0), (0, DFF - P)))
    brp = jnp.pad(br.reshape(1, P), ((0, 7), (0, DFF - P)))
    out_pad = pl.pallas_call(
        _moe_body,
        grid=(B,),
        in_specs=[
            pl.BlockSpec((1, S, C), lambda b: (b, 0, 0)),
            pl.BlockSpec((P, C, DFF), lambda b: (0, 0, 0)),
            pl.BlockSpec((P, DFF), lambda b: (0, 0)),
            pl.BlockSpec((P, C, DFF), lambda b: (0, 0, 0)),
            pl.BlockSpec((P, DFF), lambda b: (0, 0)),
            pl.BlockSpec((C, DFF), lambda b: (0, 0)),
            pl.BlockSpec((8, DFF), lambda b: (0, 0)),
        ],
        out_specs=pl.BlockSpec((1, S, 2 * C), lambda b: (b, 0, 0)),
        out_shape=jax.ShapeDtypeStruct((B, S, 2 * C), jnp.float32),
        compiler_params=pltpu.CompilerParams(
            dimension_semantics=("arbitrary",),
        ),
    )(u_t, W1, b1, W2T, b2p, Wrp, brp)
    return out_pad[:, :, :C]
